# baseline (speedup 1.0000x reference)
import functools

import jax
import jax.numpy as jnp
import numpy as np
from jax.experimental import pallas as pl
from jax.experimental.pallas import tpu as pltpu

N = 10000
E = 320000
D_IN = 128
OUT = 128
H1 = 8
MSG_DIM = 16
T_DIM = 100
EDGE_DIM = MSG_DIM + T_DIM
D_MID = H1 * OUT

BE = 2000


def _encode_body(rel_t_ref, tw_ref, tb_ref, msg_ref, out_ref):
    rel_t = rel_t_ref[...]
    tw = tw_ref[...]
    tb = tb_ref[...]
    enc = jnp.cos(rel_t * tw + tb)
    out_ref[...] = jnp.concatenate([enc, msg_ref[...]], axis=1)


def _edge_encode(rel_t, time_w, time_b, msg):
    grid = (E // BE,)
    return pl.pallas_call(
        _encode_body,
        grid=grid,
        in_specs=[
            pl.BlockSpec((BE, 1), lambda i: (i, 0)),
            pl.BlockSpec((1, T_DIM), lambda i: (0, 0)),
            pl.BlockSpec((1, T_DIM), lambda i: (0, 0)),
            pl.BlockSpec((BE, MSG_DIM), lambda i: (i, 0)),
        ],
        out_specs=pl.BlockSpec((BE, EDGE_DIM), lambda i: (i, 0)),
        out_shape=jax.ShapeDtypeStruct((E, EDGE_DIM), jnp.float32),
    )(rel_t.reshape(E, 1), time_w.reshape(1, T_DIM), time_b.reshape(1, T_DIM), msg)


def _seg_softmax(alpha, dst, n):
    amax = jax.ops.segment_max(alpha, dst, num_segments=n)
    amax = jnp.where(jnp.isfinite(amax), amax, 0.0)
    a = jnp.exp(alpha - amax[dst])
    denom = jax.ops.segment_sum(a, dst, num_segments=n)
    return a / (denom[dst] + 1e-16)


def _conv(x, src, dst, eattr, Wq, bq, Wk, bk, Wv, bv, We, be, Ws, bs, H, C, concat):
    n = x.shape[0]
    q = (x @ Wq + bq).reshape(n, H, C)
    k = (x @ Wk + bk).reshape(n, H, C)
    v = (x @ Wv + bv).reshape(n, H, C)
    e = (eattr @ We + be).reshape(-1, H, C)
    kj = k[src] + e
    alpha = jnp.sum(q[dst] * kj, axis=-1) / np.sqrt(C)
    alpha = _seg_softmax(alpha, dst, n)
    m = (v[src] + e) * alpha[:, :, None]
    out = jax.ops.segment_sum(m, dst, num_segments=n)
    if concat:
        out = out.reshape(n, H * C)
    else:
        out = out.mean(axis=1)
    return out + (x @ Ws + bs)


def kernel(x, last_update, edge_index, t, msg, time_w, time_b,
           Wq1, bq1, Wk1, bk1, Wv1, bv1, We1, be1, Ws1, bs1,
           Wq2, bq2, Wk2, bk2, Wv2, bv2, We2, be2, Ws2, bs2):
    src = edge_index[0]
    dst = edge_index[1]
    rel_t = (last_update[src] - t).astype(jnp.float32)
    eattr = _edge_encode(rel_t, time_w, time_b, msg)
    h = jax.nn.relu(_conv(x, src, dst, eattr, Wq1, bq1, Wk1, bk1, Wv1, bv1,
                          We1, be1, Ws1, bs1, H1, OUT, True))
    out = jax.nn.relu(_conv(h, src, dst, eattr, Wq2, bq2, Wk2, bk2, Wv2, bv2,
                            We2, be2, Ws2, bs2, 1, OUT, False))
    return out


# speedup vs baseline: 1.0320x; 1.0320x over previous
import functools

import jax
import jax.numpy as jnp
import numpy as np
from jax import lax
from jax.experimental import pallas as pl
from jax.experimental.pallas import tpu as pltpu
from jax.experimental.pallas import tpu_sc as plsc

N = 10000
E = 320000
D_IN = 128
OUT = 128
H1 = 8
MSG_DIM = 16
T_DIM = 100
EDGE_DIM = MSG_DIM + T_DIM
D_MID = H1 * OUT

NC, NS = 2, 16
NW = NC * NS
EB = E // NW
B1 = 16
PH = 16

BE = 2000


def _encode_body(rel_t_ref, tw_ref, tb_ref, msg_ref, out_ref):
    rel_t = rel_t_ref[...]
    enc = jnp.cos(rel_t * tw_ref[...] + tb_ref[...])
    one = jnp.ones((rel_t.shape[0], 1), jnp.float32)
    pad = jnp.zeros((rel_t.shape[0], 128 - EDGE_DIM - 1), jnp.float32)
    out_ref[...] = jnp.concatenate([enc, msg_ref[...], one, pad], axis=1)


def _edge_encode(rel_t, time_w, time_b, msg):
    return pl.pallas_call(
        _encode_body,
        grid=(E // BE,),
        in_specs=[
            pl.BlockSpec((BE, 1), lambda i: (i, 0)),
            pl.BlockSpec((1, T_DIM), lambda i: (0, 0)),
            pl.BlockSpec((1, T_DIM), lambda i: (0, 0)),
            pl.BlockSpec((BE, MSG_DIM), lambda i: (i, 0)),
        ],
        out_specs=pl.BlockSpec((BE, 128), lambda i: (i, 0)),
        out_shape=jax.ShapeDtypeStruct((E, 128), jnp.float32),
    )(rel_t.reshape(E, 1), time_w.reshape(1, T_DIM), time_b.reshape(1, T_DIM),
      msg)


_GDN = lax.GatherDimensionNumbers(
    offset_dims=(), collapsed_slice_dims=(0,), start_index_map=(0,))


def _permute16(x, perm):
    return lax.gather(x, perm[:, None], dimension_numbers=_GDN,
                      slice_sizes=(1,),
                      mode=lax.GatherScatterMode.PROMISE_IN_BOUNDS)


def _hsum16(x, lane):
    for s in (8, 4, 2, 1):
        x = x + _permute16(x, lane ^ s)
    return x

def _pass1_body(qg_hbm, k_hbm, ea_hbm, src_hbm, dst_hbm, zeros_hbm,
                p_hbm, den_hbm,
                si_v, di_v, qg_v, k_v, ea_v, p2_v, pd_v, den_sh, sem0, sem1,
                *, H, C, GOFF):
    cid = lax.axis_index("c")
    sid = lax.axis_index("s")
    wid = sid * NC + cid
    base = wid * EB

    @pl.when(sid == 0)
    def _():
        pltpu.sync_copy(zeros_hbm, den_sh)

    plsc.subcore_barrier()
    for j in range(B1):
        for i in range(128 // 16):
            p2_v[j, pl.ds(i * 16, 16)] = jnp.zeros((16,), jnp.float32)

    def batch(b, carry):
        off = base + b * B1
        pltpu.sync_copy(src_hbm.at[pl.ds(off, B1)], si_v)
        pltpu.sync_copy(dst_hbm.at[pl.ds(off, B1)], di_v)
        cp0 = pltpu.async_copy(qg_hbm.at[di_v], qg_v, sem0)
        cp1 = pltpu.async_copy(k_hbm.at[si_v], k_v, sem1)
        pltpu.sync_copy(ea_hbm.at[pl.ds(off, B1)], ea_v)
        cp0.wait()
        cp1.wait()

        lane = lax.iota(jnp.int32, 16)

        def edge(j, c2):
            ea = [ea_v[j, pl.ds(i * 16, 16)] for i in range(8)]
            row = jnp.zeros((16,), jnp.float32)
            for h in range(H):
                acc = qg_v[j, pl.ds(h * C, 16)] * k_v[j, pl.ds(h * C, 16)]
                for i in range(1, C // 16):
                    acc = acc + (qg_v[j, pl.ds(h * C + i * 16, 16)]
                                 * k_v[j, pl.ds(h * C + i * 16, 16)])
                for i in range(8):
                    acc = acc + qg_v[j, pl.ds(GOFF + h * 128 + i * 16, 16)] * ea[i]
                row = jnp.where(lane == h, _hsum16(acc, lane), row)
            pe = jnp.exp(row)
            p2_v[j, pl.ds(0, 16)] = pe
            pd_v[j, :] = pe
            return c2

        lax.fori_loop(0, B1, edge, 0, unroll=False)
        pltpu.sync_copy(pd_v, den_sh.at[di_v], add=True)
        pltpu.sync_copy(p2_v, p_hbm.at[pl.ds(off, B1)])
        return carry

    lax.fori_loop(0, EB // B1, batch, 0, unroll=False)
    plsc.subcore_barrier()

    @pl.when(sid == 0)
    def _():
        pltpu.sync_copy(den_sh, den_hbm.at[cid])


def _sc_pass1(qg, kt, ea, src, dst, H, C):
    n = qg.shape[0]
    goff = H * C
    w = goff + H * 128
    mesh = plsc.VectorSubcoreMesh(core_axis_name="c", subcore_axis_name="s",
                                  num_cores=NC, num_subcores=NS)
    f = pl.kernel(
        functools.partial(_pass1_body, H=H, C=C, GOFF=goff),
        out_type=(jax.ShapeDtypeStruct((E, 128), jnp.float32),
                  jax.ShapeDtypeStruct((NC, n, PH), jnp.float32)),
        mesh=mesh,
        scratch_types=[
            pltpu.VMEM((B1,), jnp.int32),
            pltpu.VMEM((B1,), jnp.int32),
            pltpu.VMEM((B1, w), jnp.float32),
            pltpu.VMEM((B1, goff), jnp.float32),
            pltpu.VMEM((B1, 128), jnp.float32),
            pltpu.VMEM((B1, 128), jnp.float32),
            pltpu.VMEM((B1, PH), jnp.float32),
            pltpu.VMEM_SHARED((n, PH), jnp.float32),
            pltpu.SemaphoreType.DMA,
            pltpu.SemaphoreType.DMA,
        ],
    )
    zeros = jnp.zeros((n, PH), jnp.float32)
    return f(qg, kt, ea, src, dst, zeros)


def _pass2_body(v_hbm, ea_hbm, p_hbm, dinv_hbm, sd_hbm, dst_hbm,
                zeros_hbm, acc_hbm,
                dst_blk, todo, giv_v, sv_buf, dl_buf, sd_buf,
                v_buf, ea_buf, p_buf, dinv_buf, w_buf, o_buf, idxs,
                acc_sh, sem0, sem1, sem2, sem3,
                *, H, C, WR, CH, NCHUNK):
    cid = lax.axis_index("c")
    sid = lax.axis_index("s")
    esl = E // NS
    base2 = sid * esl
    cht = CH // NS
    DBLK = 2000
    TCAP = 1024 if NCHUNK > 4 else 8192
    lane = lax.iota(jnp.int32, 16)

    def run_chunk(ci, carry):
        c = cid + 2 * ci
        cbase = c * CH
        rpn0 = WR // 64
        pltpu.sync_copy(zeros_hbm.at[pl.ds(sid * cht * rpn0, cht * rpn0)],
                        acc_sh.at[pl.ds(sid * cht * rpn0, cht * rpn0)])
        plsc.subcore_barrier()

        def scan_blk(bb, cnt0):
          pltpu.sync_copy(dst_hbm.at[pl.ds(base2 + bb * DBLK, DBLK)], dst_blk)

          def scan(b, cnt):
            d16 = dst_blk[pl.ds(b * 16, 16)]
            dloc = d16 - cbase
            mi = jnp.where(dloc >= 0, 1, 0) * jnp.where(dloc < CH, 1, 0)
            ksum = _hsum16(mi, lane)
            k = ksum[0]

            @pl.when(k > 0)
            def _():
                kk = lane + (1 - mi) * 16
                vv = bb * DBLK + b * 16 + lane
                for blk in (2, 4, 8, 16):
                    for j in (8, 4, 2, 1):
                        if j >= blk:
                            continue
                        partner = lane ^ j
                        pk = _permute16(kk, partner)
                        pv = _permute16(vv, partner)
                        lowv = jnp.where((lane & j) == 0, 1, 0)
                        upv = jnp.where((lane & blk) == 0, 1, 0)
                        ltv = jnp.where(pk < kk, 1, 0)
                        gtv = jnp.where(pk > kk, 1, 0)
                        takev = (lowv * upv + (1 - lowv) * (1 - upv)
                                 ) * ltv + (lowv * (1 - upv)
                                            + (1 - lowv) * upv) * gtv
                        kk = jnp.where(takev == 1, pk, kk)
                        vv = jnp.where(takev == 1, pv, vv)
                todo[pl.ds(jnp.minimum(cnt, TCAP), 16)] = vv
            return jnp.minimum(cnt + k, TCAP)

          return lax.fori_loop(0, DBLK // 16, scan, cnt0, unroll=False)

        cnt = lax.fori_loop(0, esl // DBLK, scan_blk, jnp.int32(0),
                            unroll=False)
        cnt_vec = jnp.full((16,), cnt, jnp.int32)

        def proc(tb, carry2):
            ids = todo[pl.ds(tb * 16, 16)]
            valid = (tb * 16 + lane) < cnt_vec
            ids = jnp.where(valid, ids, 0)
            giv_v[...] = ids + base2
            cpS = pltpu.async_copy(sd_hbm.at[giv_v], sd_buf, sem0)
            cp1 = pltpu.async_copy(ea_hbm.at[giv_v], ea_buf, sem1)
            cp2 = pltpu.async_copy(p_hbm.at[giv_v], p_buf, sem2)
            cpS.wait()
            src_vec = jnp.zeros((16,), jnp.int32)
            dst_vec = jnp.zeros((16,), jnp.int32)
            for j in range(16):
                rj = sd_buf[j, pl.ds(0, 16)]
                src_vec = jnp.where(lane == j, _permute16(rj, lane * 0),
                                    src_vec)
                dst_vec = jnp.where(lane == j, _permute16(rj, lane * 0 + 1),
                                    dst_vec)
            sv_buf[...] = src_vec
            dl_buf[...] = dst_vec
            dlvec = jnp.clip(dst_vec - cbase, 0, CH - 1)
            cp0 = pltpu.async_copy(v_hbm.at[sv_buf], v_buf, sem0)
            cp3 = pltpu.async_copy(dinv_hbm.at[dl_buf], dinv_buf, sem3)
            cp2.wait()
            cp3.wait()
            vmask = jnp.where(valid, 1.0, 0.0).astype(jnp.float32)
            for j in range(16):
                wmj = jnp.squeeze(lax.slice(vmask, (j,), (j + 1,)))
                w_buf[j, :] = (p_buf[j, pl.ds(0, 16)]
                               * dinv_buf[j, pl.ds(0, 16)]
                               * jnp.full((16,), wmj, jnp.float32))
            cp0.wait()
            cp1.wait()
            rpn = WR // 64

            def edge(j, c3):
                eaj = [ea_buf[j, pl.ds(i * 16, 16)] for i in range(8)]
                wrow = w_buf[j, :]
                for h in range(H):
                    wh = jnp.full((16,), wrow[h], jnp.float32)
                    for i in range(C // 16):
                        q = h * C + i * 16
                        o_buf[(q // 64) * 16 + j, pl.ds((q % 64 // 16) * 16,
                                                        16)] = (
                            wh * v_buf[j, pl.ds(q, 16)])
                    for i in range(8):
                        q = H * C + h * 128 + i * 16
                        o_buf[(q // 64) * 16 + j, pl.ds((q % 64 // 16) * 16,
                                                        16)] = wh * eaj[i]
                return c3

            lax.fori_loop(0, 16, edge, 0, unroll=False)
            for t in range(rpn):
                idxs[t, :] = dlvec * rpn + t
            cps = []
            for t in range(rpn):
                cps.append(pltpu.async_copy(o_buf.at[pl.ds(t * 16, 16)],
                                            acc_sh.at[idxs.at[t]], sem3,
                                            add=True))
                if t >= 8:
                    cps[t - 8].wait()
            for cp in cps[max(0, rpn - 8):]:
                cp.wait()
            return carry2

        nb = (cnt + 15) // 16
        lax.fori_loop(0, nb, proc, 0, unroll=False)
        plsc.subcore_barrier()
        pltpu.sync_copy(
            acc_sh.at[pl.ds(sid * cht * rpn0, cht * rpn0)],
            acc_hbm.at[pl.ds((cbase + sid * cht) * rpn0, cht * rpn0)])
        plsc.subcore_barrier()
        return carry

    lax.fori_loop(0, NCHUNK // NC, run_chunk, 0, unroll=False)


def _sc_pass2(vt, ea, p, dinv, sd, dst, H, C, CH, NCHUNK):
    wr = H * C + H * 128
    ntot = CH * NCHUNK
    mesh = plsc.VectorSubcoreMesh(core_axis_name="c", subcore_axis_name="s",
                                  num_cores=NC, num_subcores=NS)
    rpn = wr // 64
    f = pl.kernel(
        functools.partial(_pass2_body, H=H, C=C, WR=wr, CH=CH, NCHUNK=NCHUNK),
        out_type=jax.ShapeDtypeStruct((ntot * rpn, 64), jnp.float32),
        mesh=mesh,
        scratch_types=[
            pltpu.VMEM((2000,), jnp.int32),
            pltpu.VMEM(((1024 if NCHUNK > 4 else 8192) + 16,), jnp.int32),
            pltpu.VMEM((16,), jnp.int32),
            pltpu.VMEM((16,), jnp.int32),
            pltpu.VMEM((16,), jnp.int32),
            pltpu.VMEM((16, 128), jnp.int32),
            pltpu.VMEM((16, H * C), jnp.float32),
            pltpu.VMEM((16, 128), jnp.float32),
            pltpu.VMEM((16, 128), jnp.float32),
            pltpu.VMEM((16, 128), jnp.float32),
            pltpu.VMEM((16, PH), jnp.float32),
            pltpu.VMEM((rpn * 16, 64), jnp.float32),
            pltpu.VMEM((rpn, 16), jnp.int32),
            pltpu.VMEM_SHARED((CH * rpn, 64), jnp.float32),
            pltpu.SemaphoreType.DMA,
            pltpu.SemaphoreType.DMA,
            pltpu.SemaphoreType.DMA,
            pltpu.SemaphoreType.DMA,
        ],
    )
    zeros = jnp.zeros((CH * rpn, 64), jnp.float32)
    acc = f(vt, ea, p, dinv, sd, dst, zeros)
    return acc.reshape(ntot, wr)


BN = 400


def _proj_body(x_ref, wq_ref, bq_ref, wk_ref, bk_ref, wv_ref, bv_ref,
               ws_ref, bs_ref, wet_ref, qg_ref, kt_ref, vt_ref, sk_ref,
               *, H, C, scale):
    xb = x_ref[...]
    q = (jnp.dot(xb, wq_ref[...], preferred_element_type=jnp.float32)
         + bq_ref[...]) * scale
    qg_ref[:, pl.ds(0, H * C)] = q
    for h in range(H):
        g = jnp.dot(q[:, h * C:(h + 1) * C], wet_ref[h],
                    preferred_element_type=jnp.float32)
        qg_ref[:, pl.ds(H * C + h * 128, 128)] = g
    kt_ref[...] = jnp.dot(xb, wk_ref[...],
                          preferred_element_type=jnp.float32) + bk_ref[...]
    vt_ref[...] = jnp.dot(xb, wv_ref[...],
                          preferred_element_type=jnp.float32) + bv_ref[...]
    sk_ref[...] = jnp.dot(xb, ws_ref[...],
                          preferred_element_type=jnp.float32) + bs_ref[...]


def _tc_proj(x, Wq, bq, Wk, bk, Wv, bv, Ws, bs, We, be, H, C):
    n, d = x.shape
    hc = H * C
    WeT = We.reshape(EDGE_DIM, H, C).transpose(1, 2, 0)
    WeT = jnp.concatenate(
        [WeT, be.reshape(H, C)[:, :, None],
         jnp.zeros((H, C, 128 - EDGE_DIM - 1), jnp.float32)], axis=2)
    wspec = lambda shape: pl.BlockSpec(shape, lambda i: tuple(0 for _ in shape))
    return pl.pallas_call(
        functools.partial(_proj_body, H=H, C=C, scale=1.0 / np.sqrt(C)),
        grid=(n // BN,),
        in_specs=[
            pl.BlockSpec((BN, d), lambda i: (i, 0)),
            wspec((d, hc)), wspec((1, hc)),
            wspec((d, hc)), wspec((1, hc)),
            wspec((d, hc)), wspec((1, hc)),
            wspec((d, hc)), wspec((1, hc)),
            wspec((H, C, 128)),
        ],
        out_specs=[
            pl.BlockSpec((BN, hc + H * 128), lambda i: (i, 0)),
            pl.BlockSpec((BN, hc), lambda i: (i, 0)),
            pl.BlockSpec((BN, hc), lambda i: (i, 0)),
            pl.BlockSpec((BN, hc), lambda i: (i, 0)),
        ],
        out_shape=[
            jax.ShapeDtypeStruct((n, hc + H * 128), jnp.float32),
            jax.ShapeDtypeStruct((n, hc), jnp.float32),
            jax.ShapeDtypeStruct((n, hc), jnp.float32),
            jax.ShapeDtypeStruct((n, hc), jnp.float32),
        ],
    )(x, Wq, bq.reshape(1, hc), Wk, bk.reshape(1, hc), Wv, bv.reshape(1, hc),
      Ws, bs.reshape(1, hc), WeT)


def _post_body(acc_ref, wem_ref, sk_ref, out_ref, *, H, C, mean):
    hc = H * C
    sk = sk_ref[...]
    scale = 1.0 / H if mean else 1.0
    for h in range(H):
        accV = acc_ref[:, pl.ds(h * C, C)]
        accE = acc_ref[:, pl.ds(hc + h * 128, 128)]
        m = accV + jnp.dot(accE, wem_ref[h],
                           preferred_element_type=jnp.float32)
        if mean:
            if h == 0:
                out_ref[...] = m * scale
            else:
                out_ref[...] = out_ref[...] + m * scale
        else:
            out_ref[:, pl.ds(h * C, C)] = m
    out_ref[...] = jnp.maximum(out_ref[...] + sk, 0.0)


def _tc_post(acc, We, be, sk, H, C, mean):
    n = sk.shape[0]
    hc = H * C
    wr = hc + H * 128
    oc = C if mean else hc
    Wr = We.reshape(EDGE_DIM, H, C).transpose(1, 0, 2)
    WeM = jnp.concatenate(
        [Wr, be.reshape(H, C)[:, None, :],
         jnp.zeros((H, 128 - EDGE_DIM - 1, C), jnp.float32)], axis=1)
    wspec = lambda shape: pl.BlockSpec(shape, lambda i: tuple(0 for _ in shape))
    return pl.pallas_call(
        functools.partial(_post_body, H=H, C=C, mean=mean),
        grid=(n // BN,),
        in_specs=[
            pl.BlockSpec((BN, wr), lambda i: (i, 0)),
            wspec((H, 128, C)),
            pl.BlockSpec((BN, oc), lambda i: (i, 0)),
        ],
        out_specs=pl.BlockSpec((BN, oc), lambda i: (i, 0)),
        out_shape=jax.ShapeDtypeStruct((n, oc), jnp.float32),
    )(acc, WeM, sk)


def _conv_sc(x, src, dst, ea128, Wq, bq, Wk, bk, Wv, bv, We, be, Ws, bs,
             H, C, concat):
    n = x.shape[0]
    qg, kt, vt, sk = _tc_proj(x, Wq, bq, Wk, bk, Wv, bv, Ws, bs, We, be, H, C)
    qs = qg[:, :H * C].reshape(n, H, C)
    e = (ea128[:, :EDGE_DIM] @ We + be).reshape(E, H, C)
    kj = kt.reshape(n, H, C)[src] + e
    alpha = jnp.sum(qs[dst] * kj, axis=-1)
    amax = jax.ops.segment_max(alpha, dst, num_segments=n)
    amax = jnp.where(jnp.isfinite(amax), amax, 0.0)
    a = jnp.exp(alpha - amax[dst])
    denom = jax.ops.segment_sum(a, dst, num_segments=n)
    w = a / (denom[dst] + 1e-16)
    m = (vt.reshape(n, H, C)[src] + e) * w[:, :, None]
    out = jax.ops.segment_sum(m, dst, num_segments=n)
    out = out.reshape(n, H * C) if concat else out.mean(axis=1)
    return jax.nn.relu(out + sk)


def kernel(x, last_update, edge_index, t, msg, time_w, time_b,
           Wq1, bq1, Wk1, bk1, Wv1, bv1, We1, be1, Ws1, bs1,
           Wq2, bq2, Wk2, bk2, Wv2, bv2, We2, be2, Ws2, bs2):
    src = edge_index[0]
    dst = edge_index[1]
    rel_t = (last_update[src] - t).astype(jnp.float32)
    ea128 = _edge_encode(rel_t, time_w, time_b, msg)
    h = jax.nn.relu(_conv_sc(x, src, dst, ea128, Wq1, bq1, Wk1, bk1, Wv1, bv1,
                             We1, be1, Ws1, bs1, H1, OUT, True))
    out = jax.nn.relu(_conv_sc(h, src, dst, ea128, Wq2, bq2, Wk2, bk2, Wv2,
                               bv2, We2, be2, Ws2, bs2, 1, OUT, False))
    return out


# speedup vs baseline: 1.1015x; 1.0674x over previous
import functools

import jax
import jax.numpy as jnp
import numpy as np
from jax import lax
from jax.experimental import pallas as pl
from jax.experimental.pallas import tpu as pltpu
from jax.experimental.pallas import tpu_sc as plsc

N = 10000
E = 320000
D_IN = 128
OUT = 128
H1 = 8
MSG_DIM = 16
T_DIM = 100
EDGE_DIM = MSG_DIM + T_DIM
D_MID = H1 * OUT

NC, NS = 2, 16
NW = NC * NS
EB = E // NW
B1 = 16
PH = 16

BE = 2000


def _encode_body(rel_t_ref, tw_ref, tb_ref, msg_ref, out_ref):
    rel_t = rel_t_ref[...]
    enc = jnp.cos(rel_t * tw_ref[...] + tb_ref[...])
    one = jnp.ones((rel_t.shape[0], 1), jnp.float32)
    pad = jnp.zeros((rel_t.shape[0], 128 - EDGE_DIM - 1), jnp.float32)
    out_ref[...] = jnp.concatenate([enc, msg_ref[...], one, pad], axis=1)


def _edge_encode(rel_t, time_w, time_b, msg):
    return pl.pallas_call(
        _encode_body,
        grid=(E // BE,),
        in_specs=[
            pl.BlockSpec((BE, 1), lambda i: (i, 0)),
            pl.BlockSpec((1, T_DIM), lambda i: (0, 0)),
            pl.BlockSpec((1, T_DIM), lambda i: (0, 0)),
            pl.BlockSpec((BE, MSG_DIM), lambda i: (i, 0)),
        ],
        out_specs=pl.BlockSpec((BE, 128), lambda i: (i, 0)),
        out_shape=jax.ShapeDtypeStruct((E, 128), jnp.float32),
    )(rel_t.reshape(E, 1), time_w.reshape(1, T_DIM), time_b.reshape(1, T_DIM),
      msg)


def _p1lite_body(qg_hbm, k_hbm, ea_hbm, src_hbm, dst_hbm, p_hbm,
                 si_v, di_v, qg_v, k_v, ea_v, p2_v, sem0, sem1,
                 *, H, C, GOFF):
    cid = lax.axis_index("c")
    sid = lax.axis_index("s")
    wid = sid * NC + cid
    base = wid * EB
    for j in range(B1):
        for i in range(128 // 16):
            p2_v[j, pl.ds(i * 16, 16)] = jnp.zeros((16,), jnp.float32)

    def batch(b, carry):
        off = base + b * B1
        pltpu.sync_copy(src_hbm.at[pl.ds(off, B1)], si_v)
        pltpu.sync_copy(dst_hbm.at[pl.ds(off, B1)], di_v)
        cp0 = pltpu.async_copy(qg_hbm.at[di_v], qg_v, sem0)
        cp1 = pltpu.async_copy(k_hbm.at[si_v], k_v, sem1)
        pltpu.sync_copy(ea_hbm.at[pl.ds(off, B1)], ea_v)
        cp0.wait()
        cp1.wait()
        lane = lax.iota(jnp.int32, 16)

        def edge(j, c2):
            ea = [ea_v[j, pl.ds(i * 16, 16)] for i in range(8)]
            row = jnp.zeros((16,), jnp.float32)
            for h in range(H):
                acc = qg_v[j, pl.ds(h * C, 16)] * k_v[j, pl.ds(h * C, 16)]
                for i in range(1, C // 16):
                    acc = acc + (qg_v[j, pl.ds(h * C + i * 16, 16)]
                                 * k_v[j, pl.ds(h * C + i * 16, 16)])
                for i in range(8):
                    acc = acc + (qg_v[j, pl.ds(GOFF + h * 128 + i * 16, 16)]
                                 * ea[i])
                row = jnp.where(lane == h, _hsum16(acc, lane), row)
            p2_v[j, pl.ds(0, 16)] = jnp.exp(row)
            return c2

        lax.fori_loop(0, B1, edge, 0, unroll=False)
        pltpu.sync_copy(p2_v, p_hbm.at[pl.ds(off, B1)])
        return carry

    lax.fori_loop(0, EB // B1, batch, 0, unroll=False)


def _sc_p1lite(qg, kt, ea, src, dst, H, C):
    goff = H * C
    w = goff + H * 128
    mesh = plsc.VectorSubcoreMesh(core_axis_name="c", subcore_axis_name="s",
                                  num_cores=NC, num_subcores=NS)
    f = pl.kernel(
        functools.partial(_p1lite_body, H=H, C=C, GOFF=goff),
        out_type=jax.ShapeDtypeStruct((E, 128), jnp.float32),
        mesh=mesh,
        scratch_types=[
            pltpu.VMEM((B1,), jnp.int32),
            pltpu.VMEM((B1,), jnp.int32),
            pltpu.VMEM((B1, w), jnp.float32),
            pltpu.VMEM((B1, goff), jnp.float32),
            pltpu.VMEM((B1, 128), jnp.float32),
            pltpu.VMEM((B1, 128), jnp.float32),
            pltpu.SemaphoreType.DMA,
            pltpu.SemaphoreType.DMA,
        ],
    )
    return f(qg, kt, ea, src, dst)

_GDN = lax.GatherDimensionNumbers(
    offset_dims=(), collapsed_slice_dims=(0,), start_index_map=(0,))


def _permute16(x, perm):
    return lax.gather(x, perm[:, None], dimension_numbers=_GDN,
                      slice_sizes=(1,),
                      mode=lax.GatherScatterMode.PROMISE_IN_BOUNDS)


def _hsum16(x, lane):
    for s in (8, 4, 2, 1):
        x = x + _permute16(x, lane ^ s)
    return x

def _pass1_body(qg_hbm, k_hbm, ea_hbm, src_hbm, dst_hbm, zeros_hbm,
                p_hbm, den_hbm,
                si_v, di_v, qg_v, k_v, ea_v, p2_v, pd_v, den_sh, sem0, sem1,
                *, H, C, GOFF):
    cid = lax.axis_index("c")
    sid = lax.axis_index("s")
    wid = sid * NC + cid
    base = wid * EB

    @pl.when(sid == 0)
    def _():
        pltpu.sync_copy(zeros_hbm, den_sh)

    plsc.subcore_barrier()
    for j in range(B1):
        for i in range(128 // 16):
            p2_v[j, pl.ds(i * 16, 16)] = jnp.zeros((16,), jnp.float32)

    def batch(b, carry):
        off = base + b * B1
        pltpu.sync_copy(src_hbm.at[pl.ds(off, B1)], si_v)
        pltpu.sync_copy(dst_hbm.at[pl.ds(off, B1)], di_v)
        cp0 = pltpu.async_copy(qg_hbm.at[di_v], qg_v, sem0)
        cp1 = pltpu.async_copy(k_hbm.at[si_v], k_v, sem1)
        pltpu.sync_copy(ea_hbm.at[pl.ds(off, B1)], ea_v)
        cp0.wait()
        cp1.wait()

        lane = lax.iota(jnp.int32, 16)

        def edge(j, c2):
            ea = [ea_v[j, pl.ds(i * 16, 16)] for i in range(8)]
            row = jnp.zeros((16,), jnp.float32)
            for h in range(H):
                acc = qg_v[j, pl.ds(h * C, 16)] * k_v[j, pl.ds(h * C, 16)]
                for i in range(1, C // 16):
                    acc = acc + (qg_v[j, pl.ds(h * C + i * 16, 16)]
                                 * k_v[j, pl.ds(h * C + i * 16, 16)])
                for i in range(8):
                    acc = acc + qg_v[j, pl.ds(GOFF + h * 128 + i * 16, 16)] * ea[i]
                row = jnp.where(lane == h, _hsum16(acc, lane), row)
            pe = jnp.exp(row)
            p2_v[j, pl.ds(0, 16)] = pe
            pd_v[j, :] = pe
            return c2

        lax.fori_loop(0, B1, edge, 0, unroll=False)
        pltpu.sync_copy(pd_v, den_sh.at[di_v], add=True)
        pltpu.sync_copy(p2_v, p_hbm.at[pl.ds(off, B1)])
        return carry

    lax.fori_loop(0, EB // B1, batch, 0, unroll=False)
    plsc.subcore_barrier()

    @pl.when(sid == 0)
    def _():
        pltpu.sync_copy(den_sh, den_hbm.at[cid])


def _sc_pass1(qg, kt, ea, src, dst, H, C):
    n = qg.shape[0]
    goff = H * C
    w = goff + H * 128
    mesh = plsc.VectorSubcoreMesh(core_axis_name="c", subcore_axis_name="s",
                                  num_cores=NC, num_subcores=NS)
    f = pl.kernel(
        functools.partial(_pass1_body, H=H, C=C, GOFF=goff),
        out_type=(jax.ShapeDtypeStruct((E, 128), jnp.float32),
                  jax.ShapeDtypeStruct((NC, n, PH), jnp.float32)),
        mesh=mesh,
        scratch_types=[
            pltpu.VMEM((B1,), jnp.int32),
            pltpu.VMEM((B1,), jnp.int32),
            pltpu.VMEM((B1, w), jnp.float32),
            pltpu.VMEM((B1, goff), jnp.float32),
            pltpu.VMEM((B1, 128), jnp.float32),
            pltpu.VMEM((B1, 128), jnp.float32),
            pltpu.VMEM((B1, PH), jnp.float32),
            pltpu.VMEM_SHARED((n, PH), jnp.float32),
            pltpu.SemaphoreType.DMA,
            pltpu.SemaphoreType.DMA,
        ],
    )
    zeros = jnp.zeros((n, PH), jnp.float32)
    return f(qg, kt, ea, src, dst, zeros)


def _pass2_body(v_hbm, ea_hbm, p_hbm, dinv_hbm, sd_hbm, dst_hbm,
                zeros_hbm, acc_hbm,
                dst_blk, todo, giv_v, sv_buf, dl_buf, sd_buf,
                v_buf, ea_buf, p_buf, dinv_buf, w_buf, o_buf, idxs,
                acc_sh, sem0, sem1, sem2, sem3,
                *, H, C, WR, CH, NCHUNK):
    cid = lax.axis_index("c")
    sid = lax.axis_index("s")
    esl = E // NS
    base2 = sid * esl
    cht = CH // NS
    DBLK = 2000
    TCAP = 1024 if NCHUNK > 4 else 8192
    lane = lax.iota(jnp.int32, 16)

    def run_chunk(ci, carry):
        c = cid + 2 * ci
        cbase = c * CH
        rpn0 = WR // 64
        pltpu.sync_copy(zeros_hbm.at[pl.ds(sid * cht * rpn0, cht * rpn0)],
                        acc_sh.at[pl.ds(sid * cht * rpn0, cht * rpn0)])
        plsc.subcore_barrier()

        def scan_blk(bb, cnt0):
          pltpu.sync_copy(dst_hbm.at[pl.ds(base2 + bb * DBLK, DBLK)], dst_blk)

          def scan(b, cnt):
            d16 = dst_blk[pl.ds(b * 16, 16)]
            dloc = d16 - cbase
            mi = jnp.where(dloc >= 0, 1, 0) * jnp.where(dloc < CH, 1, 0)
            ksum = _hsum16(mi, lane)
            k = ksum[0]

            @pl.when(k > 0)
            def _():
                kk = lane + (1 - mi) * 16
                vv = bb * DBLK + b * 16 + lane
                for blk in (2, 4, 8, 16):
                    for j in (8, 4, 2, 1):
                        if j >= blk:
                            continue
                        partner = lane ^ j
                        pk = _permute16(kk, partner)
                        pv = _permute16(vv, partner)
                        lowv = jnp.where((lane & j) == 0, 1, 0)
                        upv = jnp.where((lane & blk) == 0, 1, 0)
                        ltv = jnp.where(pk < kk, 1, 0)
                        gtv = jnp.where(pk > kk, 1, 0)
                        takev = (lowv * upv + (1 - lowv) * (1 - upv)
                                 ) * ltv + (lowv * (1 - upv)
                                            + (1 - lowv) * upv) * gtv
                        kk = jnp.where(takev == 1, pk, kk)
                        vv = jnp.where(takev == 1, pv, vv)
                todo[pl.ds(jnp.minimum(cnt, TCAP), 16)] = vv
            return jnp.minimum(cnt + k, TCAP)

          return lax.fori_loop(0, DBLK // 16, scan, cnt0, unroll=False)

        cnt = lax.fori_loop(0, esl // DBLK, scan_blk, jnp.int32(0),
                            unroll=False)
        cnt_vec = jnp.full((16,), cnt, jnp.int32)

        def proc(tb, carry2):
            ids = todo[pl.ds(tb * 16, 16)]
            valid = (tb * 16 + lane) < cnt_vec
            ids = jnp.where(valid, ids, 0)
            giv_v[...] = ids + base2
            cpS = pltpu.async_copy(sd_hbm.at[giv_v], sd_buf, sem0)
            cp1 = pltpu.async_copy(ea_hbm.at[giv_v], ea_buf, sem1)
            cp2 = pltpu.async_copy(p_hbm.at[giv_v], p_buf, sem2)
            cpS.wait()
            src_vec = jnp.zeros((16,), jnp.int32)
            dst_vec = jnp.zeros((16,), jnp.int32)
            for j in range(16):
                rj = sd_buf[j, pl.ds(0, 16)]
                src_vec = jnp.where(lane == j, _permute16(rj, lane * 0),
                                    src_vec)
                dst_vec = jnp.where(lane == j, _permute16(rj, lane * 0 + 1),
                                    dst_vec)
            sv_buf[...] = src_vec
            dl_buf[...] = dst_vec
            dlvec = jnp.clip(dst_vec - cbase, 0, CH - 1)
            cp0 = pltpu.async_copy(v_hbm.at[sv_buf], v_buf, sem0)
            cp3 = pltpu.async_copy(dinv_hbm.at[dl_buf], dinv_buf, sem3)
            cp2.wait()
            cp3.wait()
            vmask = jnp.where(valid, 1.0, 0.0).astype(jnp.float32)
            for j in range(16):
                wmj = jnp.squeeze(lax.slice(vmask, (j,), (j + 1,)))
                w_buf[j, :] = (p_buf[j, pl.ds(0, 16)]
                               * dinv_buf[j, pl.ds(0, 16)]
                               * jnp.full((16,), wmj, jnp.float32))
            cp0.wait()
            cp1.wait()
            rpn = WR // 64

            def edge(j, c3):
                eaj = [ea_buf[j, pl.ds(i * 16, 16)] for i in range(8)]
                wrow = w_buf[j, :]
                for h in range(H):
                    wh = jnp.full((16,), wrow[h], jnp.float32)
                    for i in range(C // 16):
                        q = h * C + i * 16
                        o_buf[(q // 64) * 16 + j, pl.ds((q % 64 // 16) * 16,
                                                        16)] = (
                            wh * v_buf[j, pl.ds(q, 16)])
                    for i in range(8):
                        q = H * C + h * 128 + i * 16
                        o_buf[(q // 64) * 16 + j, pl.ds((q % 64 // 16) * 16,
                                                        16)] = wh * eaj[i]
                return c3

            lax.fori_loop(0, 16, edge, 0, unroll=False)
            for t in range(rpn):
                idxs[t, :] = dlvec * rpn + t
            cps = []
            for t in range(rpn):
                cps.append(pltpu.async_copy(o_buf.at[pl.ds(t * 16, 16)],
                                            acc_sh.at[idxs.at[t]], sem3,
                                            add=True))
                if t >= 8:
                    cps[t - 8].wait()
            for cp in cps[max(0, rpn - 8):]:
                cp.wait()
            return carry2

        nb = (cnt + 15) // 16
        lax.fori_loop(0, nb, proc, 0, unroll=False)
        plsc.subcore_barrier()
        pltpu.sync_copy(
            acc_sh.at[pl.ds(sid * cht * rpn0, cht * rpn0)],
            acc_hbm.at[pl.ds((cbase + sid * cht) * rpn0, cht * rpn0)])
        plsc.subcore_barrier()
        return carry

    lax.fori_loop(0, NCHUNK // NC, run_chunk, 0, unroll=False)


def _sc_pass2(vt, ea, p, dinv, sd, dst, H, C, CH, NCHUNK):
    wr = H * C + H * 128
    ntot = CH * NCHUNK
    mesh = plsc.VectorSubcoreMesh(core_axis_name="c", subcore_axis_name="s",
                                  num_cores=NC, num_subcores=NS)
    rpn = wr // 64
    f = pl.kernel(
        functools.partial(_pass2_body, H=H, C=C, WR=wr, CH=CH, NCHUNK=NCHUNK),
        out_type=jax.ShapeDtypeStruct((ntot * rpn, 64), jnp.float32),
        mesh=mesh,
        scratch_types=[
            pltpu.VMEM((2000,), jnp.int32),
            pltpu.VMEM(((1024 if NCHUNK > 4 else 8192) + 16,), jnp.int32),
            pltpu.VMEM((16,), jnp.int32),
            pltpu.VMEM((16,), jnp.int32),
            pltpu.VMEM((16,), jnp.int32),
            pltpu.VMEM((16, 128), jnp.int32),
            pltpu.VMEM((16, H * C), jnp.float32),
            pltpu.VMEM((16, 128), jnp.float32),
            pltpu.VMEM((16, 128), jnp.float32),
            pltpu.VMEM((16, 128), jnp.float32),
            pltpu.VMEM((16, PH), jnp.float32),
            pltpu.VMEM((rpn * 16, 64), jnp.float32),
            pltpu.VMEM((rpn, 16), jnp.int32),
            pltpu.VMEM_SHARED((CH * rpn, 64), jnp.float32),
            pltpu.SemaphoreType.DMA,
            pltpu.SemaphoreType.DMA,
            pltpu.SemaphoreType.DMA,
            pltpu.SemaphoreType.DMA,
        ],
    )
    zeros = jnp.zeros((CH * rpn, 64), jnp.float32)
    acc = f(vt, ea, p, dinv, sd, dst, zeros)
    return acc.reshape(ntot, wr)


BN = 400


def _proj_body(x_ref, wq_ref, bq_ref, wk_ref, bk_ref, wv_ref, bv_ref,
               ws_ref, bs_ref, wet_ref, qg_ref, kt_ref, vt_ref, sk_ref,
               *, H, C, scale):
    xb = x_ref[...]
    q = (jnp.dot(xb, wq_ref[...], preferred_element_type=jnp.float32)
         + bq_ref[...]) * scale
    qg_ref[:, pl.ds(0, H * C)] = q
    for h in range(H):
        g = jnp.dot(q[:, h * C:(h + 1) * C], wet_ref[h],
                    preferred_element_type=jnp.float32)
        qg_ref[:, pl.ds(H * C + h * 128, 128)] = g
    kt_ref[...] = jnp.dot(xb, wk_ref[...],
                          preferred_element_type=jnp.float32) + bk_ref[...]
    vt_ref[...] = jnp.dot(xb, wv_ref[...],
                          preferred_element_type=jnp.float32) + bv_ref[...]
    sk_ref[...] = jnp.dot(xb, ws_ref[...],
                          preferred_element_type=jnp.float32) + bs_ref[...]


def _tc_proj(x, Wq, bq, Wk, bk, Wv, bv, Ws, bs, We, be, H, C):
    n, d = x.shape
    hc = H * C
    WeT = We.reshape(EDGE_DIM, H, C).transpose(1, 2, 0)
    WeT = jnp.concatenate(
        [WeT, be.reshape(H, C)[:, :, None],
         jnp.zeros((H, C, 128 - EDGE_DIM - 1), jnp.float32)], axis=2)
    wspec = lambda shape: pl.BlockSpec(shape, lambda i: tuple(0 for _ in shape))
    return pl.pallas_call(
        functools.partial(_proj_body, H=H, C=C, scale=1.0 / np.sqrt(C)),
        grid=(n // BN,),
        in_specs=[
            pl.BlockSpec((BN, d), lambda i: (i, 0)),
            wspec((d, hc)), wspec((1, hc)),
            wspec((d, hc)), wspec((1, hc)),
            wspec((d, hc)), wspec((1, hc)),
            wspec((d, hc)), wspec((1, hc)),
            wspec((H, C, 128)),
        ],
        out_specs=[
            pl.BlockSpec((BN, hc + H * 128), lambda i: (i, 0)),
            pl.BlockSpec((BN, hc), lambda i: (i, 0)),
            pl.BlockSpec((BN, hc), lambda i: (i, 0)),
            pl.BlockSpec((BN, hc), lambda i: (i, 0)),
        ],
        out_shape=[
            jax.ShapeDtypeStruct((n, hc + H * 128), jnp.float32),
            jax.ShapeDtypeStruct((n, hc), jnp.float32),
            jax.ShapeDtypeStruct((n, hc), jnp.float32),
            jax.ShapeDtypeStruct((n, hc), jnp.float32),
        ],
    )(x, Wq, bq.reshape(1, hc), Wk, bk.reshape(1, hc), Wv, bv.reshape(1, hc),
      Ws, bs.reshape(1, hc), WeT)


def _post_body(acc_ref, wem_ref, sk_ref, out_ref, *, H, C, mean):
    hc = H * C
    sk = sk_ref[...]
    scale = 1.0 / H if mean else 1.0
    for h in range(H):
        accV = acc_ref[:, pl.ds(h * C, C)]
        accE = acc_ref[:, pl.ds(hc + h * 128, 128)]
        m = accV + jnp.dot(accE, wem_ref[h],
                           preferred_element_type=jnp.float32)
        if mean:
            if h == 0:
                out_ref[...] = m * scale
            else:
                out_ref[...] = out_ref[...] + m * scale
        else:
            out_ref[:, pl.ds(h * C, C)] = m
    out_ref[...] = jnp.maximum(out_ref[...] + sk, 0.0)


def _tc_post(acc, We, be, sk, H, C, mean):
    n = sk.shape[0]
    hc = H * C
    wr = hc + H * 128
    oc = C if mean else hc
    Wr = We.reshape(EDGE_DIM, H, C).transpose(1, 0, 2)
    WeM = jnp.concatenate(
        [Wr, be.reshape(H, C)[:, None, :],
         jnp.zeros((H, 128 - EDGE_DIM - 1, C), jnp.float32)], axis=1)
    wspec = lambda shape: pl.BlockSpec(shape, lambda i: tuple(0 for _ in shape))
    return pl.pallas_call(
        functools.partial(_post_body, H=H, C=C, mean=mean),
        grid=(n // BN,),
        in_specs=[
            pl.BlockSpec((BN, wr), lambda i: (i, 0)),
            wspec((H, 128, C)),
            pl.BlockSpec((BN, oc), lambda i: (i, 0)),
        ],
        out_specs=pl.BlockSpec((BN, oc), lambda i: (i, 0)),
        out_shape=jax.ShapeDtypeStruct((n, oc), jnp.float32),
    )(acc, WeM, sk)


def _conv_sc(x, src, dst, ea128, Wq, bq, Wk, bk, Wv, bv, We, be, Ws, bs,
             H, C, concat):
    n = x.shape[0]
    qg, kt, vt, sk = _tc_proj(x, Wq, bq, Wk, bk, Wv, bv, Ws, bs, We, be, H, C)
    a = _sc_p1lite(qg, kt, ea128, src, dst, H, C)[:, :H]
    e = (ea128[:, :EDGE_DIM] @ We + be).reshape(E, H, C)
    denom = jax.ops.segment_sum(a, dst, num_segments=n)
    w = a / (denom[dst] + 1e-16)
    m = (vt.reshape(n, H, C)[src] + e) * w[:, :, None]
    out = jax.ops.segment_sum(m, dst, num_segments=n)
    out = out.reshape(n, H * C) if concat else out.mean(axis=1)
    return jax.nn.relu(out + sk)


def kernel(x, last_update, edge_index, t, msg, time_w, time_b,
           Wq1, bq1, Wk1, bk1, Wv1, bv1, We1, be1, Ws1, bs1,
           Wq2, bq2, Wk2, bk2, Wv2, bv2, We2, be2, Ws2, bs2):
    src = edge_index[0]
    dst = edge_index[1]
    rel_t = (last_update[src] - t).astype(jnp.float32)
    ea128 = _edge_encode(rel_t, time_w, time_b, msg)
    h = jax.nn.relu(_conv_sc(x, src, dst, ea128, Wq1, bq1, Wk1, bk1, Wv1, bv1,
                             We1, be1, Ws1, bs1, H1, OUT, True))
    out = jax.nn.relu(_conv_sc(h, src, dst, ea128, Wq2, bq2, Wk2, bk2, Wv2,
                               bv2, We2, be2, Ws2, bs2, 1, OUT, False))
    return out
